# baseline (speedup 1.0000x reference)
import functools

import jax
import jax.numpy as jnp
from jax import lax
from jax.experimental import pallas as pl
from jax.experimental.pallas import tpu as pltpu
from jax.experimental.pallas import tpu_sc as plsc

NUM_EMBEDDINGS = 1000000
EMBEDDING_DIM = 64
TOTAL_VALUES = 106496

NC = 2
NS = 16
NW = NC * NS
BPW = TOTAL_VALUES // NW
R = 64
NCHUNKS = BPW // R

_mesh = plsc.VectorSubcoreMesh(core_axis_name="c", subcore_axis_name="s")


@functools.partial(
    pl.kernel,
    out_type=jax.ShapeDtypeStruct((TOTAL_VALUES, EMBEDDING_DIM), jnp.float32),
    mesh=_mesh,
    compiler_params=pltpu.CompilerParams(needs_layout_passes=False),
    scratch_types=[
        pltpu.VMEM((BPW,), jnp.int32),
        pltpu.VMEM((R * 8, EMBEDDING_DIM), jnp.float32),
        pltpu.VMEM((R, EMBEDDING_DIM), jnp.float32),
        pltpu.SemaphoreType.DMA,
        pltpu.SemaphoreType.DMA,
    ],
)
def _sc_gather(table_hbm, idx_hbm, out_hbm, idx_v, slab, stage, gsem, ssem):
    wid = lax.axis_index("s") * NC + lax.axis_index("c")
    base = wid * BPW
    pltpu.sync_copy(idx_hbm.at[pl.ds(base, BPW)], idx_v)

    lanes = lax.iota(jnp.int32, 16)

    @pl.loop(0, NCHUNKS)
    def _(c):
        cb = c * R
        waits = []
        for j in range(R // 16):
            gb = idx_v[pl.ds(cb + j * 16, 16)] & ~jnp.int32(7)
            for t in range(16):
                i = j * 16 + t
                off = pl.multiple_of(gb[t], 8)
                waits.append(pltpu.async_copy(
                    table_hbm.at[pl.ds(off, 8)],
                    slab.at[pl.ds(i * 8, 8)], gsem))
        for w in waits:
            w.wait()
        for j in range(R // 16):
            subs = idx_v[pl.ds(cb + j * 16, 16)] & 7
            for t in range(16):
                i = j * 16 + t
                d0 = jnp.full((16,), i * 8, dtype=jnp.int32) + subs[t]
                for k in range(EMBEDDING_DIM // 16):
                    vals = plsc.load_gather(slab, [d0, lanes + k * 16])
                    stage[i, pl.ds(k * 16, 16)] = vals
        off = pl.multiple_of(base + cb, R)
        pltpu.async_copy(stage, out_hbm.at[pl.ds(off, R)], ssem).wait()


def kernel(table, indices, offsets):
    del offsets
    return _sc_gather(table, indices.astype(jnp.int32))
